# baseline (speedup 1.0000x reference)
import functools

import jax
import jax.numpy as jnp
from jax import lax
from jax.experimental import pallas as pl
from jax.experimental.pallas import tpu as pltpu
from jax.experimental.pallas import tpu_sc as plsc

N = 10000
E = 320000
H = 32
F = 4 * H
NW = 32
NPT = 313
NPAD = NW * NPT


def _proj_body(nf_ref, w_ref, b_ref, o_ref):
    o_ref[...] = (
        jnp.dot(nf_ref[...], w_ref[...], preferred_element_type=jnp.float32)
        + b_ref[...]
    )


def _project(nf, w, b):
    return pl.pallas_call(
        _proj_body,
        out_shape=jax.ShapeDtypeStruct((nf.shape[0], w.shape[1]), jnp.float32),
    )(nf, w, b.reshape(1, -1))


CH_G = 1000


def _gather_body(src_hbm, dst_hbm, xs_hbm, xd_hbm, gs_hbm, gd_hbm,
                 idx_v, rows_a, rows_b, sem):
    wid = lax.axis_index("s") * 2 + lax.axis_index("c")
    epw = E // NW
    base = wid * epw

    def chunk(i, carry):
        off = base + i * CH_G
        pltpu.sync_copy(src_hbm.at[pl.ds(off, CH_G)], idx_v)
        pltpu.async_copy(xs_hbm.at[idx_v], rows_a, sem).wait()
        pltpu.sync_copy(rows_a, gs_hbm.at[pl.ds(off, CH_G)])
        pltpu.sync_copy(dst_hbm.at[pl.ds(off, CH_G)], idx_v)
        pltpu.async_copy(xd_hbm.at[idx_v], rows_b, sem).wait()
        pltpu.sync_copy(rows_b, gd_hbm.at[pl.ds(off, CH_G)])
        return carry

    lax.fori_loop(0, epw // CH_G, chunk, 0)


_gather_call = pl.kernel(
    _gather_body,
    out_type=(
        jax.ShapeDtypeStruct((E, H), jnp.float32),
        jax.ShapeDtypeStruct((E, H), jnp.float32),
    ),
    mesh=plsc.VectorSubcoreMesh(core_axis_name="c", subcore_axis_name="s"),
    compiler_params=pltpu.CompilerParams(use_tc_tiling_on_sc=False),
    scratch_types=[
        pltpu.VMEM((CH_G,), jnp.int32),
        pltpu.VMEM((CH_G, H), jnp.float32),
        pltpu.VMEM((CH_G, H), jnp.float32),
        pltpu.SemaphoreType.DMA,
    ],
)


BE = 3200


def _mlp_body(gs_ref, gd_ref, w1a_ref, w1b_ref, b1_ref, w2f_ref, b2f_ref,
              w2k_ref, b2k_ref, o1_ref, o2_ref, o3_ref, o4_ref):
    h = jnp.dot(gs_ref[...], w1a_ref[...], preferred_element_type=jnp.float32)
    h = h + jnp.dot(gd_ref[...], w1b_ref[...], preferred_element_type=jnp.float32)
    h = jnp.maximum(h + b1_ref[...], 0.0)
    kl = jnp.dot(h, w2k_ref[...], preferred_element_type=jnp.float32) + b2k_ref[...]
    gate = 1.0 / (1.0 + jnp.exp(-kl))
    f = jnp.dot(h, w2f_ref[...], preferred_element_type=jnp.float32) + b2f_ref[...]
    f = f * gate
    o1_ref[...] = f[:, 0 * H:1 * H]
    o2_ref[...] = f[:, 1 * H:2 * H]
    o3_ref[...] = f[:, 2 * H:3 * H]
    o4_ref[...] = f[:, 3 * H:4 * H]


def _edge_mlp(gs, gd, W1, b1, W2, b2):
    w1a = W1[:H]
    w1b = W1[H:]
    w2k = W2[:, :1]
    w2f = W2[:, 1:]
    b2k = b2[:1].reshape(1, 1)
    b2f = b2[1:].reshape(1, F)
    full = lambda shape: pl.BlockSpec(shape, lambda i: (0, 0))
    return pl.pallas_call(
        _mlp_body,
        grid=(E // BE,),
        in_specs=[
            pl.BlockSpec((BE, H), lambda i: (i, 0)),
            pl.BlockSpec((BE, H), lambda i: (i, 0)),
            full((H, 2 * H)),
            full((H, 2 * H)),
            full((1, 2 * H)),
            full((2 * H, F)),
            full((1, F)),
            full((2 * H, 1)),
            full((1, 1)),
        ],
        out_specs=[pl.BlockSpec((BE, H), lambda i: (i, 0))] * 4,
        out_shape=[jax.ShapeDtypeStruct((E, H), jnp.float32)] * 4,
    )(gs, gd, w1a, w1b, b1.reshape(1, 2 * H), w2f, b2f, w2k, b2k)


CHE = 6400
NCHK = E // CHE
GR = 256
NGRP = (CHE + GR - 1) // GR
CAP = CHE + GR


def _seg_body(ids_hbm, m1_hbm, m2_hbm, m3_hbm, m4_hbm, acc_hbm,
              ids_v, midx_v, mloc_v, mprev_v, gA, gB,
              r1A, r2A, r3A, r4A, r1B, r2B, r3B, r4B,
              acc_v, semA, semB):
    wid = lax.axis_index("s") * 2 + lax.axis_index("c")
    lo = wid * NPT

    zero16 = jnp.zeros((16,), jnp.float32)
    ninf16 = jnp.full((16,), -jnp.inf, jnp.float32)
    pinf16 = jnp.full((16,), jnp.inf, jnp.float32)
    npt16 = jnp.full((16,), NPT, jnp.int32)
    iota16 = lax.iota(jnp.int32, 16)

    def zi(i, c):
        midx_v[pl.ds(i * 16, 16)] = i * 16 + iota16
        mloc_v[pl.ds(i * 16, 16)] = npt16
        return c

    lax.fori_loop(0, CAP // 16, zi, 0)

    def ia(r, c):
        base = r * F
        for cc in range(8):
            if cc < 2 or cc >= 6:
                acc_v[pl.ds(base + cc * 16, 16)] = zero16
            elif cc < 4:
                acc_v[pl.ds(base + cc * 16, 16)] = ninf16
            else:
                acc_v[pl.ds(base + cc * 16, 16)] = pinf16
        return c

    lax.fori_loop(0, NPT + 1, ia, 0)

    def accumulate(rows, use_prev, gb):
        r1, r2, r3, r4 = rows

        def group16(q, c3):
            if use_prev:
                dl_vec = mprev_v[pl.ds(q * 16, 16)]
            else:
                dl_vec = mloc_v[pl.ds(gb + q * 16, 16)]
            for lane in range(16):
                dl = dl_vec[lane]
                i = q * 16 + lane
                for cc in range(8):
                    rv = (r1, r1, r2, r2, r3, r3, r4, r4)[cc]
                    r = rv[i, pl.ds((cc % 2) * 16, 16)]
                    a = acc_v[pl.ds(dl * F + cc * 16, 16)]
                    if cc < 2 or cc >= 6:
                        a = a + r
                    elif cc < 4:
                        a = jnp.maximum(a, r)
                    else:
                        a = jnp.minimum(a, r)
                    acc_v[pl.ds(dl * F + cc * 16, 16)] = a
            return c3

        lax.fori_loop(0, GR // 16, group16, 0)

    def issue4(g_ref, rows, sem):
        r1, r2, r3, r4 = rows
        cp1 = pltpu.make_async_copy(m1_hbm.at[g_ref], r1, sem)
        cp2 = pltpu.make_async_copy(m2_hbm.at[g_ref], r2, sem)
        cp3 = pltpu.make_async_copy(m3_hbm.at[g_ref], r3, sem)
        cp4 = pltpu.make_async_copy(m4_hbm.at[g_ref], r4, sem)
        cp1.start(); cp2.start(); cp3.start(); cp4.start()
        return (cp1, cp2, cp3, cp4)

    def wait4(g_ref, rows, sem):
        r1, r2, r3, r4 = rows
        pltpu.make_async_copy(m1_hbm.at[g_ref], r1, sem).wait()
        pltpu.make_async_copy(m2_hbm.at[g_ref], r2, sem).wait()
        pltpu.make_async_copy(m3_hbm.at[g_ref], r3, sem).wait()
        pltpu.make_async_copy(m4_hbm.at[g_ref], r4, sem).wait()

    def process(coff, g_cur, rows_cur, sem_cur, g_prv, rows_prv, sem_prv,
                pos_prev):
        pltpu.sync_copy(ids_hbm.at[pl.ds(coff, CHE)], ids_v)

        def vec(j, pos_vec):
            v = ids_v[pl.ds(j * 16, 16)]
            local = v - lo
            m = local.astype(jnp.uint32) < jnp.uint32(NPT)
            offs = plsc.cumsum(m.astype(jnp.int32))
            tgt = pos_vec + offs - 1
            eidx = (coff + j * 16) + iota16
            plsc.store_scatter(midx_v, [tgt], eidx, mask=m)
            plsc.store_scatter(mloc_v, [tgt], local, mask=m)
            return pos_vec + plsc.all_reduce_population_count(m)

        pos_vec = lax.fori_loop(0, CHE // 16, vec, jnp.zeros((16,), jnp.int32))
        pos = pos_vec[0]

        for k in range(GR // 16):
            plsc.store_scatter(mloc_v, [pos + k * 16 + iota16], npt16)

        for t in range(GR // 16):
            gcv = midx_v[pl.ds(t * 16, 16)]
            g_cur[pl.ds(t * 16, 16)] = gcv

        @pl.when(pos > 0)
        def _():
            issue4(g_cur, rows_cur, sem_cur)

        @pl.when(pos_prev > 0)
        def _():
            wait4(g_prv, rows_prv, sem_prv)
            accumulate(rows_prv, True, 0)

        def extra(g, c2):
            gb = g * GR

            @pl.when(gb < pos)
            def _():
                def stg(t, c4):
                    gpv = midx_v[pl.ds(gb + t * 16, 16)]
                    g_prv[pl.ds(t * 16, 16)] = gpv
                    return c4

                lax.fori_loop(0, GR // 16, stg, 0)
                issue4(g_prv, rows_prv, sem_prv)
                wait4(g_prv, rows_prv, sem_prv)
                accumulate(rows_prv, False, gb)

            return c2

        lax.fori_loop(1, NGRP, extra, 0)

        for t in range(GR // 16):
            mpv = mloc_v[pl.ds(t * 16, 16)]
            mprev_v[pl.ds(t * 16, 16)] = mpv

        return pos

    rowsA = (r1A, r2A, r3A, r4A)
    rowsB = (r1B, r2B, r3B, r4B)

    def two(k, pos_prev):
        pos_a = process(
            (2 * k) * CHE, gA, rowsA, semA, gB, rowsB, semB, pos_prev
        )
        pos_b = process(
            (2 * k + 1) * CHE, gB, rowsB, semB, gA, rowsA, semA, pos_a
        )
        return pos_b

    posf = lax.fori_loop(0, NCHK // 2, two, 0)

    @pl.when(posf > 0)
    def _():
        wait4(gB, rowsB, semB)
        accumulate(rowsB, True, 0)

    pltpu.sync_copy(
        acc_v.at[pl.ds(0, NPT * F)], acc_hbm.at[pl.ds(lo * F, NPT * F)]
    )


_seg_call = pl.kernel(
    _seg_body,
    out_type=jax.ShapeDtypeStruct((NPAD * F,), jnp.float32),
    mesh=plsc.VectorSubcoreMesh(core_axis_name="c", subcore_axis_name="s"),
    compiler_params=pltpu.CompilerParams(
        use_tc_tiling_on_sc=False, needs_layout_passes=False
    ),
    scratch_types=[
        pltpu.VMEM((CHE,), jnp.int32),
        pltpu.VMEM((CAP,), jnp.int32),
        pltpu.VMEM((CAP,), jnp.int32),
        pltpu.VMEM((GR,), jnp.int32),
        pltpu.VMEM((GR,), jnp.int32),
        pltpu.VMEM((GR,), jnp.int32),
        pltpu.VMEM((GR, H), jnp.float32),
        pltpu.VMEM((GR, H), jnp.float32),
        pltpu.VMEM((GR, H), jnp.float32),
        pltpu.VMEM((GR, H), jnp.float32),
        pltpu.VMEM((GR, H), jnp.float32),
        pltpu.VMEM((GR, H), jnp.float32),
        pltpu.VMEM((GR, H), jnp.float32),
        pltpu.VMEM((GR, H), jnp.float32),
        pltpu.VMEM(((NPT + 1) * F,), jnp.float32),
        pltpu.SemaphoreType.DMA,
        pltpu.SemaphoreType.DMA,
    ],
)


def _readout_body(x_ref, acc_ref, wr1_ref, wr2_ref, br_ref, g1a_ref, g1b_ref,
                  gb1_ref, g2_ref, gb2_ref, o_ref):
    x = x_ref[...]
    a = acc_ref[0:N, :]
    a = jnp.where(jnp.abs(a) == jnp.inf, 0.0, a)
    ncx = (
        jnp.dot(x, wr1_ref[...], preferred_element_type=jnp.float32)
        + jnp.dot(a, wr2_ref[...], preferred_element_type=jnp.float32)
        + br_ref[...]
    )
    hh = jnp.maximum(
        jnp.dot(x, g1a_ref[...], preferred_element_type=jnp.float32)
        + jnp.dot(ncx, g1b_ref[...], preferred_element_type=jnp.float32)
        + gb1_ref[...],
        0.0,
    )
    o_ref[...] = (
        jnp.dot(hh, g2_ref[...], preferred_element_type=jnp.float32)
        + gb2_ref[...]
    )


def _readout(x, acc, Wr, br, W1, b1, W2, b2):
    return pl.pallas_call(
        _readout_body,
        out_shape=jax.ShapeDtypeStruct((N, H), jnp.float32),
    )(x, acc, Wr[:H], Wr[H:], br.reshape(1, H), W1[:H], W1[H:],
      b1.reshape(1, H), W2, b2.reshape(1, H))


def kernel(nf_gc, nf_gs, ei_s2c, ei_c2s, W_gc, b_gc, W_gs, b_gs, Ws1, bs1,
           Ws2, bs2, Wrs, brs, Wc1, bc1, Wc2, bc2, Wrc, brc, gcW1, gcb1,
           gcW2, gcb2, gsW1, gsb1, gsW2, gsb2):
    x_gc = _project(nf_gc, W_gc, b_gc)
    x_gs = _project(nf_gs, W_gs, b_gs)

    def direction(x_src, x_dst, ei, W1, b1, W2, b2):
        src = ei[0]
        dst = ei[1]
        gsrc, gdst = _gather_call(src, dst, x_src, x_dst)
        m1, m2, m3, m4 = _edge_mlp(gsrc, gdst, W1, b1, W2, b2)
        acc = _seg_call(dst, m1, m2, m3, m4)
        return acc.reshape(NPAD, F)

    acc_c = direction(x_gs, x_gc, ei_s2c, Ws1, bs1, Ws2, bs2)
    acc_s = direction(x_gc, x_gs, ei_c2s, Wc1, bc1, Wc2, bc2)
    out_fc = _readout(x_gc, acc_c, Wrs, brs, gcW1, gcb1, gcW2, gcb2)
    out_fs = _readout(x_gs, acc_s, Wrc, brc, gsW1, gsb1, gsW2, gsb2)
    return out_fc, out_fs


# speedup vs baseline: 1.0522x; 1.0522x over previous
import functools

import jax
import jax.numpy as jnp
from jax import lax
from jax.experimental import pallas as pl
from jax.experimental.pallas import tpu as pltpu
from jax.experimental.pallas import tpu_sc as plsc

N = 10000
E = 320000
H = 32
F = 4 * H
NW = 32
NPT = 313
NPAD = NW * NPT


def _proj_body(nf_ref, w_ref, b_ref, o_ref):
    o_ref[...] = (
        jnp.dot(nf_ref[...], w_ref[...], preferred_element_type=jnp.float32)
        + b_ref[...]
    )


def _project(nf, w, b):
    return pl.pallas_call(
        _proj_body,
        out_shape=jax.ShapeDtypeStruct((nf.shape[0], w.shape[1]), jnp.float32),
    )(nf, w, b.reshape(1, -1))


CH_G = 1000


def _gather_body(src_hbm, dst_hbm, xs_hbm, xd_hbm, gs_hbm, gd_hbm,
                 idx_v, rows_a, rows_b, sem):
    wid = lax.axis_index("s") * 2 + lax.axis_index("c")
    epw = E // NW
    base = wid * epw

    def chunk(i, carry):
        off = base + i * CH_G
        pltpu.sync_copy(src_hbm.at[pl.ds(off, CH_G)], idx_v)
        pltpu.async_copy(xs_hbm.at[idx_v], rows_a, sem).wait()
        pltpu.sync_copy(rows_a, gs_hbm.at[pl.ds(off, CH_G)])
        pltpu.sync_copy(dst_hbm.at[pl.ds(off, CH_G)], idx_v)
        pltpu.async_copy(xd_hbm.at[idx_v], rows_b, sem).wait()
        pltpu.sync_copy(rows_b, gd_hbm.at[pl.ds(off, CH_G)])
        return carry

    lax.fori_loop(0, epw // CH_G, chunk, 0)


_gather_call = pl.kernel(
    _gather_body,
    out_type=(
        jax.ShapeDtypeStruct((E, H), jnp.float32),
        jax.ShapeDtypeStruct((E, H), jnp.float32),
    ),
    mesh=plsc.VectorSubcoreMesh(core_axis_name="c", subcore_axis_name="s"),
    compiler_params=pltpu.CompilerParams(use_tc_tiling_on_sc=False),
    scratch_types=[
        pltpu.VMEM((CH_G,), jnp.int32),
        pltpu.VMEM((CH_G, H), jnp.float32),
        pltpu.VMEM((CH_G, H), jnp.float32),
        pltpu.SemaphoreType.DMA,
    ],
)


BE = 3200


def _mlp_body(gs_ref, gd_ref, w1a_ref, w1b_ref, b1_ref, w2f_ref, b2f_ref,
              w2k_ref, b2k_ref, o1_ref, o2_ref, o3_ref, o4_ref):
    h = jnp.dot(gs_ref[...], w1a_ref[...], preferred_element_type=jnp.float32)
    h = h + jnp.dot(gd_ref[...], w1b_ref[...], preferred_element_type=jnp.float32)
    h = jnp.maximum(h + b1_ref[...], 0.0)
    kl = jnp.dot(h, w2k_ref[...], preferred_element_type=jnp.float32) + b2k_ref[...]
    gate = 1.0 / (1.0 + jnp.exp(-kl))
    f = jnp.dot(h, w2f_ref[...], preferred_element_type=jnp.float32) + b2f_ref[...]
    f = f * gate
    o1_ref[...] = f[:, 0 * H:1 * H]
    o2_ref[...] = f[:, 1 * H:2 * H]
    o3_ref[...] = f[:, 2 * H:3 * H]
    o4_ref[...] = f[:, 3 * H:4 * H]


def _edge_mlp(gs, gd, W1, b1, W2, b2):
    w1a = W1[:H]
    w1b = W1[H:]
    w2k = W2[:, :1]
    w2f = W2[:, 1:]
    b2k = b2[:1].reshape(1, 1)
    b2f = b2[1:].reshape(1, F)
    full = lambda shape: pl.BlockSpec(shape, lambda i: (0, 0))
    return pl.pallas_call(
        _mlp_body,
        grid=(E // BE,),
        in_specs=[
            pl.BlockSpec((BE, H), lambda i: (i, 0)),
            pl.BlockSpec((BE, H), lambda i: (i, 0)),
            full((H, 2 * H)),
            full((H, 2 * H)),
            full((1, 2 * H)),
            full((2 * H, F)),
            full((1, F)),
            full((2 * H, 1)),
            full((1, 1)),
        ],
        out_specs=[pl.BlockSpec((BE, H), lambda i: (i, 0))] * 4,
        out_shape=[jax.ShapeDtypeStruct((E, H), jnp.float32)] * 4,
    )(gs, gd, w1a, w1b, b1.reshape(1, 2 * H), w2f, b2f, w2k, b2k)


CHE = 6400
NCHK = E // CHE
GR = 256
NGRP = (CHE + GR - 1) // GR
CAP = CHE + GR


def _seg_body(ids_hbm, m1_hbm, m2_hbm, m3_hbm, m4_hbm, acc_hbm,
              ids_v, midx_v, mloc_v, mprev_v, gA, gB,
              r1A, r2A, r3A, r4A, r1B, r2B, r3B, r4B,
              acc_v, semA, semB):
    wid = lax.axis_index("s") * 2 + lax.axis_index("c")
    lo = wid * NPT

    zero16 = jnp.zeros((16,), jnp.float32)
    ninf16 = jnp.full((16,), -jnp.inf, jnp.float32)
    pinf16 = jnp.full((16,), jnp.inf, jnp.float32)
    npt16 = jnp.full((16,), NPT, jnp.int32)
    iota16 = lax.iota(jnp.int32, 16)

    def zi(i, c):
        midx_v[pl.ds(i * 16, 16)] = i * 16 + iota16
        mloc_v[pl.ds(i * 16, 16)] = npt16
        return c

    lax.fori_loop(0, CAP // 16, zi, 0)

    def ia(r, c):
        base = r * F
        for cc in range(8):
            if cc < 2 or cc >= 6:
                acc_v[pl.ds(base + cc * 16, 16)] = zero16
            elif cc < 4:
                acc_v[pl.ds(base + cc * 16, 16)] = ninf16
            else:
                acc_v[pl.ds(base + cc * 16, 16)] = pinf16
        return c

    lax.fori_loop(0, NPT + 1, ia, 0)

    def accumulate(rows, use_prev, gb):
        r1, r2, r3, r4 = rows

        def group16(q, c3):
            if use_prev:
                dl_vec = mprev_v[pl.ds(q * 16, 16)]
            else:
                dl_vec = mloc_v[pl.ds(gb + q * 16, 16)]
            for lane in range(16):
                dl = dl_vec[lane]
                i = q * 16 + lane
                for cc in range(8):
                    rv = (r1, r1, r2, r2, r3, r3, r4, r4)[cc]
                    r = rv[i, pl.ds((cc % 2) * 16, 16)]
                    a = acc_v[pl.ds(dl * F + cc * 16, 16)]
                    if cc < 2 or cc >= 6:
                        a = a + r
                    elif cc < 4:
                        a = jnp.maximum(a, r)
                    else:
                        a = jnp.minimum(a, r)
                    acc_v[pl.ds(dl * F + cc * 16, 16)] = a
            return c3

        lax.fori_loop(0, GR // 16, group16, 0)

    def issue4(g_ref, rows, sem):
        r1, r2, r3, r4 = rows
        cp1 = pltpu.make_async_copy(m1_hbm.at[g_ref], r1, sem)
        cp2 = pltpu.make_async_copy(m2_hbm.at[g_ref], r2, sem)
        cp3 = pltpu.make_async_copy(m3_hbm.at[g_ref], r3, sem)
        cp4 = pltpu.make_async_copy(m4_hbm.at[g_ref], r4, sem)
        cp1.start(); cp2.start(); cp3.start(); cp4.start()
        return (cp1, cp2, cp3, cp4)

    def wait4(g_ref, rows, sem):
        r1, r2, r3, r4 = rows
        pltpu.make_async_copy(m1_hbm.at[g_ref], r1, sem).wait()
        pltpu.make_async_copy(m2_hbm.at[g_ref], r2, sem).wait()
        pltpu.make_async_copy(m3_hbm.at[g_ref], r3, sem).wait()
        pltpu.make_async_copy(m4_hbm.at[g_ref], r4, sem).wait()

    def process(coff, g_cur, rows_cur, sem_cur, g_prv, rows_prv, sem_prv,
                pos_prev):
        pltpu.sync_copy(ids_hbm.at[pl.ds(coff, CHE)], ids_v)

        def vec(j, pos_vec):
            base = j * 64
            vs = []
            for u in range(4):
                v = ids_v[pl.ds(base + u * 16, 16)]
                local = v - lo
                m = local.astype(jnp.uint32) < jnp.uint32(NPT)
                vs.append((local, m, plsc.cumsum(m.astype(jnp.int32)),
                           plsc.all_reduce_population_count(m)))
            for u, (local, m, offs, cnt) in enumerate(vs):
                tgt = pos_vec + offs - 1
                eidx = (coff + base + u * 16) + iota16
                plsc.store_scatter(midx_v, [tgt], eidx, mask=m)
                plsc.store_scatter(mloc_v, [tgt], local, mask=m)
                pos_vec = pos_vec + cnt
            return pos_vec

        pos_vec = lax.fori_loop(0, CHE // 64, vec, jnp.zeros((16,), jnp.int32))
        pos = pos_vec[0]

        for k in range(GR // 16):
            plsc.store_scatter(mloc_v, [pos + k * 16 + iota16], npt16)

        for t in range(GR // 16):
            gcv = midx_v[pl.ds(t * 16, 16)]
            g_cur[pl.ds(t * 16, 16)] = gcv

        @pl.when(pos > 0)
        def _():
            issue4(g_cur, rows_cur, sem_cur)

        @pl.when(pos_prev > 0)
        def _():
            wait4(g_prv, rows_prv, sem_prv)
            accumulate(rows_prv, True, 0)

        def extra(g, c2):
            gb = g * GR

            @pl.when(gb < pos)
            def _():
                def stg(t, c4):
                    gpv = midx_v[pl.ds(gb + t * 16, 16)]
                    g_prv[pl.ds(t * 16, 16)] = gpv
                    return c4

                lax.fori_loop(0, GR // 16, stg, 0)
                issue4(g_prv, rows_prv, sem_prv)
                wait4(g_prv, rows_prv, sem_prv)
                accumulate(rows_prv, False, gb)

            return c2

        lax.fori_loop(1, NGRP, extra, 0)

        for t in range(GR // 16):
            mpv = mloc_v[pl.ds(t * 16, 16)]
            mprev_v[pl.ds(t * 16, 16)] = mpv

        return pos

    rowsA = (r1A, r2A, r3A, r4A)
    rowsB = (r1B, r2B, r3B, r4B)

    def two(k, pos_prev):
        pos_a = process(
            (2 * k) * CHE, gA, rowsA, semA, gB, rowsB, semB, pos_prev
        )
        pos_b = process(
            (2 * k + 1) * CHE, gB, rowsB, semB, gA, rowsA, semA, pos_a
        )
        return pos_b

    posf = lax.fori_loop(0, NCHK // 2, two, 0)

    @pl.when(posf > 0)
    def _():
        wait4(gB, rowsB, semB)
        accumulate(rowsB, True, 0)

    pltpu.sync_copy(
        acc_v.at[pl.ds(0, NPT * F)], acc_hbm.at[pl.ds(lo * F, NPT * F)]
    )


_seg_call = pl.kernel(
    _seg_body,
    out_type=jax.ShapeDtypeStruct((NPAD * F,), jnp.float32),
    mesh=plsc.VectorSubcoreMesh(core_axis_name="c", subcore_axis_name="s"),
    compiler_params=pltpu.CompilerParams(
        use_tc_tiling_on_sc=False, needs_layout_passes=False
    ),
    scratch_types=[
        pltpu.VMEM((CHE,), jnp.int32),
        pltpu.VMEM((CAP,), jnp.int32),
        pltpu.VMEM((CAP,), jnp.int32),
        pltpu.VMEM((GR,), jnp.int32),
        pltpu.VMEM((GR,), jnp.int32),
        pltpu.VMEM((GR,), jnp.int32),
        pltpu.VMEM((GR, H), jnp.float32),
        pltpu.VMEM((GR, H), jnp.float32),
        pltpu.VMEM((GR, H), jnp.float32),
        pltpu.VMEM((GR, H), jnp.float32),
        pltpu.VMEM((GR, H), jnp.float32),
        pltpu.VMEM((GR, H), jnp.float32),
        pltpu.VMEM((GR, H), jnp.float32),
        pltpu.VMEM((GR, H), jnp.float32),
        pltpu.VMEM(((NPT + 1) * F,), jnp.float32),
        pltpu.SemaphoreType.DMA,
        pltpu.SemaphoreType.DMA,
    ],
)


def _readout_body(x_ref, acc_ref, wr1_ref, wr2_ref, br_ref, g1a_ref, g1b_ref,
                  gb1_ref, g2_ref, gb2_ref, o_ref):
    x = x_ref[...]
    a = acc_ref[0:N, :]
    a = jnp.where(jnp.abs(a) == jnp.inf, 0.0, a)
    ncx = (
        jnp.dot(x, wr1_ref[...], preferred_element_type=jnp.float32)
        + jnp.dot(a, wr2_ref[...], preferred_element_type=jnp.float32)
        + br_ref[...]
    )
    hh = jnp.maximum(
        jnp.dot(x, g1a_ref[...], preferred_element_type=jnp.float32)
        + jnp.dot(ncx, g1b_ref[...], preferred_element_type=jnp.float32)
        + gb1_ref[...],
        0.0,
    )
    o_ref[...] = (
        jnp.dot(hh, g2_ref[...], preferred_element_type=jnp.float32)
        + gb2_ref[...]
    )


def _readout(x, acc, Wr, br, W1, b1, W2, b2):
    return pl.pallas_call(
        _readout_body,
        out_shape=jax.ShapeDtypeStruct((N, H), jnp.float32),
    )(x, acc, Wr[:H], Wr[H:], br.reshape(1, H), W1[:H], W1[H:],
      b1.reshape(1, H), W2, b2.reshape(1, H))


def kernel(nf_gc, nf_gs, ei_s2c, ei_c2s, W_gc, b_gc, W_gs, b_gs, Ws1, bs1,
           Ws2, bs2, Wrs, brs, Wc1, bc1, Wc2, bc2, Wrc, brc, gcW1, gcb1,
           gcW2, gcb2, gsW1, gsb1, gsW2, gsb2):
    x_gc = _project(nf_gc, W_gc, b_gc)
    x_gs = _project(nf_gs, W_gs, b_gs)

    def direction(x_src, x_dst, ei, W1, b1, W2, b2):
        src = ei[0]
        dst = ei[1]
        gsrc, gdst = _gather_call(src, dst, x_src, x_dst)
        m1, m2, m3, m4 = _edge_mlp(gsrc, gdst, W1, b1, W2, b2)
        acc = _seg_call(dst, m1, m2, m3, m4)
        return acc.reshape(NPAD, F)

    acc_c = direction(x_gs, x_gc, ei_s2c, Ws1, bs1, Ws2, bs2)
    acc_s = direction(x_gc, x_gs, ei_c2s, Wc1, bc1, Wc2, bc2)
    out_fc = _readout(x_gc, acc_c, Wrs, brs, gcW1, gcb1, gcW2, gcb2)
    out_fs = _readout(x_gs, acc_s, Wrc, brc, gsW1, gsb1, gsW2, gsb2)
    return out_fc, out_fs
